# baseline (speedup 1.0000x reference)
import functools

import jax
import jax.numpy as jnp
from jax import lax
from jax.experimental import pallas as pl
from jax.experimental.pallas import tpu as pltpu

N0 = 10000
N1 = 20000
E = 640000
H = 128
HH = 64

RB0 = 2000
RB1 = 2000


def _proj_body(x_ref, w_ref, b_ref, o_ref):
    x = x_ref[...]
    w = w_ref[0, :, :]
    b = b_ref[0, :, :]
    xb = x.astype(jnp.bfloat16)
    wb = w.astype(jnp.bfloat16)
    o_ref[0, :, :] = jnp.dot(xb, wb, preferred_element_type=jnp.float32) + b


def _proj_in(x, W, b):
    F = x.shape[1]
    Wr = W.reshape(F, 2, HH).transpose(1, 0, 2)
    br = b.reshape(2, 1, HH)
    grid = (N0 // RB0, 2)
    return pl.pallas_call(
        _proj_body,
        grid=grid,
        in_specs=[
            pl.BlockSpec((RB0, F), lambda i, c: (i, 0)),
            pl.BlockSpec((1, F, HH), lambda i, c: (c, 0, 0)),
            pl.BlockSpec((1, 1, HH), lambda i, c: (c, 0, 0)),
        ],
        out_specs=pl.BlockSpec((1, RB0, HH), lambda i, c: (c, i, 0)),
        out_shape=jax.ShapeDtypeStruct((2, N0, HH), jnp.float32),
    )(x, Wr, br)


def _lin_body(x_ref, w_ref, b_ref, o_ref):
    x0 = x_ref[0, :, :]
    x1 = x_ref[1, :, :]
    w0 = w_ref[0, 0, :, :]
    w1 = w_ref[0, 1, :, :]
    b = b_ref[0, :, :]
    acc = jnp.dot(x0.astype(jnp.bfloat16), w0.astype(jnp.bfloat16),
                  preferred_element_type=jnp.float32)
    acc = acc + jnp.dot(x1.astype(jnp.bfloat16), w1.astype(jnp.bfloat16),
                        preferred_element_type=jnp.float32)
    o_ref[0, :, :] = acc + b


def _linear(x2, W, b):
    Wr = W.reshape(2, HH, 2, HH).transpose(2, 0, 1, 3)
    br = b.reshape(2, 1, HH)
    grid = (N0 // RB0, 2)
    return pl.pallas_call(
        _lin_body,
        grid=grid,
        in_specs=[
            pl.BlockSpec((2, RB0, HH), lambda i, c: (0, i, 0)),
            pl.BlockSpec((1, 2, HH, HH), lambda i, c: (c, 0, 0, 0)),
            pl.BlockSpec((1, 1, HH), lambda i, c: (c, 0, 0)),
        ],
        out_specs=pl.BlockSpec((1, RB0, HH), lambda i, c: (c, i, 0)),
        out_shape=jax.ShapeDtypeStruct((2, N0, HH), jnp.float32),
    )(x2, Wr, br)


def _combine_body(a_ref, m_ref, r_ref, o_ref, s_ref, *, relu):
    i = pl.program_id(1)
    a = a_ref[0, :, :]
    m = m_ref[0, :, :]
    r = r_ref[...]
    x = a + m * r
    if relu:
        x = jnp.maximum(x, 0.0)
    o_ref[0, :, :] = x

    @pl.when(i == 0)
    def _():
        s_ref[...] = jnp.zeros_like(s_ref)
    s_ref[0, 0, :] += jnp.sum(x, axis=0)


def _combine(a2, msg2, rdeg, relu):
    grid = (2, N0 // RB0)
    return pl.pallas_call(
        functools.partial(_combine_body, relu=relu),
        grid=grid,
        in_specs=[
            pl.BlockSpec((1, RB0, HH), lambda c, i: (c, i, 0)),
            pl.BlockSpec((1, RB0, HH), lambda c, i: (c, i, 0)),
            pl.BlockSpec((RB0, 1), lambda c, i: (i, 0)),
        ],
        out_specs=[
            pl.BlockSpec((1, RB0, HH), lambda c, i: (c, i, 0)),
            pl.BlockSpec((1, 1, HH), lambda c, i: (c, 0, 0)),
        ],
        out_shape=[
            jax.ShapeDtypeStruct((2, N0, HH), jnp.float32),
            jax.ShapeDtypeStruct((2, 1, HH), jnp.float32),
        ],
    )(a2, msg2, rdeg)


def _colsum_body(x_ref, s_ref):
    i = pl.program_id(1)

    @pl.when(i == 0)
    def _():
        s_ref[...] = jnp.zeros_like(s_ref)
    s_ref[0, 0, :] += jnp.sum(x_ref[0, :, :], axis=0)


def _colsum(x2, n, rb):
    grid = (2, n // rb)
    return pl.pallas_call(
        _colsum_body,
        grid=grid,
        in_specs=[pl.BlockSpec((1, rb, HH), lambda c, i: (c, i, 0))],
        out_specs=pl.BlockSpec((1, 1, HH), lambda c, i: (c, 0, 0)),
        out_shape=jax.ShapeDtypeStruct((2, 1, HH), jnp.float32),
    )(x2)


def _head_body(s0_ref, s1_ref, w0_ref, w1_ref, b_ref, o_ref):
    s0 = s0_ref[...]
    s1 = s1_ref[...]
    w0 = w0_ref[...]
    w1 = w1_ref[...]
    w0b = w0.astype(jnp.bfloat16).astype(jnp.float32)
    w1b = w1.astype(jnp.bfloat16).astype(jnp.float32)
    m0 = jnp.sum(s0 * w0b, axis=(0, 2)) / N0
    m1 = jnp.sum(s1 * w1b, axis=(0, 2)) / N1
    o_ref[...] = (m0 + m1).reshape(1, 1) + b_ref[...]


def _head(s0, s1, Wo0, Wo1, bo0, bo1):
    w0 = Wo0.reshape(2, 1, HH)
    w1 = Wo1.reshape(2, 1, HH)
    b = (bo0 + bo1).reshape(1, 1)
    out = pl.pallas_call(
        _head_body,
        in_specs=[
            pl.BlockSpec((2, 1, HH), lambda: (0, 0, 0)),
            pl.BlockSpec((2, 1, HH), lambda: (0, 0, 0)),
            pl.BlockSpec((2, 1, HH), lambda: (0, 0, 0)),
            pl.BlockSpec((2, 1, HH), lambda: (0, 0, 0)),
            pl.BlockSpec((1, 1), lambda: (0, 0)),
        ],
        out_specs=pl.BlockSpec((1, 1), lambda: (0, 0)),
        out_shape=jax.ShapeDtypeStruct((1, 1), jnp.float32),
    )(s0, s1, w0, w1, b)
    return out.reshape(1)


def _sc_layer(a2, v_idx, e_idx, need_deg):
    a = jnp.concatenate([a2[0], a2[1]], axis=1)
    x1 = jax.ops.segment_sum(a[v_idx], e_idx, num_segments=N1)
    msg = jax.ops.segment_sum(x1[e_idx], v_idx, num_segments=N0)
    x1_2 = jnp.stack([x1[:, :HH], x1[:, HH:]])
    msg2 = jnp.stack([msg[:, :HH], msg[:, HH:]])
    deg = None
    if need_deg:
        deg = jax.ops.segment_sum(jnp.ones((E,), jnp.float32), v_idx,
                                  num_segments=N0)
    return x1_2, msg2, deg


def kernel(x_0, x_1, vertex_idx, hyperedge_idx,
           W0_in, b0_in, W1_in, b1_in, Wl0, bl0, Wl1, bl1,
           Wo0, bo0, Wo1, bo1):
    v_idx = vertex_idx.astype(jnp.int32)
    e_idx = hyperedge_idx.astype(jnp.int32)

    h0 = _proj_in(x_0, W0_in, b0_in)

    a1 = _linear(h0, Wl0, bl0)
    _, msg1, deg = _sc_layer(a1, v_idx, e_idx, True)
    rdeg = (1.0 / jnp.maximum(deg, 1.0)).reshape(N0, 1)
    x0_1, _ = _combine(a1, msg1, rdeg, relu=True)

    a2 = _linear(x0_1, Wl1, bl1)
    x1_2, msg2, _ = _sc_layer(a2, v_idx, e_idx, False)
    x0_2, s0 = _combine(a2, msg2, rdeg, relu=False)

    s1 = _colsum(x1_2, N1, RB1)
    return _head(s0, s1, Wo0, Wo1, bo0, bo1)


# speedup vs baseline: 3.8683x; 3.8683x over previous
import functools

import jax
import jax.numpy as jnp
from jax import lax
from jax.experimental import pallas as pl
from jax.experimental.pallas import tpu as pltpu
from jax.experimental.pallas import tpu_sc as plsc

N0 = 10000
N1 = 20000
E = 640000
H = 128

RB0 = 2000
RB1 = 2000


def _mm_body(x_ref, w_ref, b_ref, o_ref):
    x = x_ref[...].astype(jnp.bfloat16)
    w = w_ref[...].astype(jnp.bfloat16)
    o_ref[...] = (jnp.dot(x, w, preferred_element_type=jnp.float32)
                  + b_ref[...])


def _matmul(x, W, b):
    n, f = x.shape
    return pl.pallas_call(
        _mm_body,
        grid=(n // RB0,),
        in_specs=[
            pl.BlockSpec((RB0, f), lambda i: (i, 0)),
            pl.BlockSpec((f, H), lambda i: (0, 0)),
            pl.BlockSpec((1, H), lambda i: (0, 0)),
        ],
        out_specs=pl.BlockSpec((RB0, H), lambda i: (i, 0)),
        out_shape=jax.ShapeDtypeStruct((n, H), jnp.float32),
    )(x, W, b.reshape(1, H))


def _combine_body(a_ref, m_ref, d_ref, o_ref, s_ref, *, relu):
    i = pl.program_id(0)
    deg = d_ref[:, 0:1]
    r = 1.0 / jnp.maximum(deg, 1.0)
    x = a_ref[...] + m_ref[...] * r
    if relu:
        x = jnp.maximum(x, 0.0)
    o_ref[...] = x

    @pl.when(i == 0)
    def _():
        s_ref[...] = jnp.zeros_like(s_ref)
    s_ref[...] += jnp.sum(x, axis=0, keepdims=True)


def _combine(a, msg, degp, relu):
    return pl.pallas_call(
        functools.partial(_combine_body, relu=relu),
        grid=(N0 // RB0,),
        in_specs=[
            pl.BlockSpec((RB0, H), lambda i: (i, 0)),
            pl.BlockSpec((RB0, H), lambda i: (i, 0)),
            pl.BlockSpec((RB0, H), lambda i: (i, 0)),
        ],
        out_specs=[
            pl.BlockSpec((RB0, H), lambda i: (i, 0)),
            pl.BlockSpec((1, H), lambda i: (0, 0)),
        ],
        out_shape=[
            jax.ShapeDtypeStruct((N0, H), jnp.float32),
            jax.ShapeDtypeStruct((1, H), jnp.float32),
        ],
    )(a, msg, degp)


def _colsum_body(x_ref, s_ref):
    i = pl.program_id(0)

    @pl.when(i == 0)
    def _():
        s_ref[...] = jnp.zeros_like(s_ref)
    s_ref[...] += jnp.sum(x_ref[...], axis=0, keepdims=True)


def _colsum(x, rb):
    n = x.shape[0]
    return pl.pallas_call(
        _colsum_body,
        grid=(n // rb,),
        in_specs=[pl.BlockSpec((rb, H), lambda i: (i, 0))],
        out_specs=pl.BlockSpec((1, H), lambda i: (0, 0)),
        out_shape=jax.ShapeDtypeStruct((1, H), jnp.float32),
    )(x)


def _head_body(s0_ref, s1_ref, w0_ref, w1_ref, b_ref, o_ref):
    w0 = w0_ref[...].astype(jnp.bfloat16).astype(jnp.float32)
    w1 = w1_ref[...].astype(jnp.bfloat16).astype(jnp.float32)
    m0 = jnp.sum(s0_ref[...] * w0) / N0
    m1 = jnp.sum(s1_ref[...] * w1) / N1
    o_ref[...] = (m0 + m1).reshape(1, 1) + b_ref[...]


def _head(s0, s1, Wo0, Wo1, bo0, bo1):
    out = pl.pallas_call(
        _head_body,
        in_specs=[pl.BlockSpec((1, H), lambda: (0, 0))] * 4 +
                 [pl.BlockSpec((1, 1), lambda: (0, 0))],
        out_specs=pl.BlockSpec((1, 1), lambda: (0, 0)),
        out_shape=jax.ShapeDtypeStruct((1, 1), jnp.float32),
    )(s0, s1, Wo0.reshape(1, H), Wo1.reshape(1, H),
      (bo0 + bo1).reshape(1, 1))
    return out.reshape(1)


CW = 128
NCHUNK = E // CW
NTILES = 16
CPT = (NCHUNK + NTILES - 1) // NTILES
X1H = N1 // 2
MSGH = N0 // 2


def _sc_layer_kernel(do_deg):
    mesh = plsc.VectorSubcoreMesh(core_axis_name="c", subcore_axis_name="s")
    out_type = [
        jax.ShapeDtypeStruct((N1, H), jnp.float32),
        jax.ShapeDtypeStruct((2, MSGH, H), jnp.float32),
        jax.ShapeDtypeStruct((2, MSGH, H), jnp.float32),
    ]
    DEG0 = MSGH + 8
    scratch = [
        pltpu.VMEM_SHARED((X1H + 16, H), jnp.float32),
        pltpu.VMEM((1, CW), jnp.int32),
        pltpu.VMEM((1, CW), jnp.int32),
        pltpu.VMEM((1, CW), jnp.int32),
        pltpu.VMEM((CW, H), jnp.float32),
        pltpu.VMEM((CW, H), jnp.float32),
        pltpu.SemaphoreType.DMA,
    ]

    @functools.partial(pl.kernel, out_type=out_type, mesh=mesh,
                       scratch_types=scratch)
    def k(a0, vidx, eidx, z, ones_h, x1_out, msg_out, deg_out,
          sh, vbuf, ebuf, lbuf, rows, onesbuf, sem):
        c = lax.axis_index("c")
        s = lax.axis_index("s")

        @pl.when(s < 10)
        def _():
            pltpu.sync_copy(z, sh.at[pl.ds(s * 1000, 1000)])
        if do_deg:
            pltpu.sync_copy(ones_h, onesbuf)
        plsc.subcore_barrier()

        e_lo = c * X1H
        v_lo = c * MSGH

        def p1(i, carry):
            j = s + i * NTILES

            @pl.when(j < NCHUNK)
            def _():
                pltpu.sync_copy(vidx.at[j], vbuf)
                pltpu.sync_copy(eidx.at[j], ebuf)
                for t in range(CW // 16):
                    sl = pl.ds(t * 16, 16)
                    le = ebuf[0, sl] - e_lo
                    ok = (le >= 0) & (le < X1H)
                    lbuf[0, sl] = jnp.where(ok, le, X1H + (t % 8))
                pltpu.async_copy(a0.at[vbuf.at[0]], rows, sem).wait()
                pltpu.sync_copy(rows, sh.at[lbuf.at[0]], add=True)
            return carry

        lax.fori_loop(0, CPT, p1, 0)
        plsc.subcore_barrier()

        @pl.when(s < 10)
        def _():
            pltpu.sync_copy(sh.at[pl.ds(s * 1000, 1000)],
                            x1_out.at[pl.ds(c * X1H + s * 1000, 1000)])
            pltpu.sync_copy(z, sh.at[pl.ds(s * 1000, 1000)])
        if do_deg:
            @pl.when(s == 10)
            def _():
                pltpu.sync_copy(z.at[pl.ds(0, 16)],
                                sh.at[pl.ds(10000, 16)])
        plsc.subcore_barrier()

        def p2(i, carry):
            j = s + i * NTILES

            @pl.when(j < NCHUNK)
            def _():
                pltpu.sync_copy(vidx.at[j], vbuf)
                pltpu.sync_copy(eidx.at[j], ebuf)
                for t in range(CW // 16):
                    sl = pl.ds(t * 16, 16)
                    lv = vbuf[0, sl] - v_lo
                    ok = (lv >= 0) & (lv < MSGH)
                    lbuf[0, sl] = jnp.where(ok, lv, MSGH + (t % 8))
                pltpu.async_copy(x1_out.at[ebuf.at[0]], rows, sem).wait()
                pltpu.sync_copy(rows, sh.at[lbuf.at[0]], add=True)
                if do_deg:
                    for t in range(CW // 16):
                        sl = pl.ds(t * 16, 16)
                        lbuf[0, sl] = lbuf[0, sl] + DEG0
                    pltpu.sync_copy(onesbuf, sh.at[lbuf.at[0]], add=True)
            return carry

        lax.fori_loop(0, CPT, p2, 0)
        plsc.subcore_barrier()

        @pl.when(s < 5)
        def _():
            pltpu.sync_copy(sh.at[pl.ds(s * 1000, 1000)],
                            msg_out.at[c].at[pl.ds(s * 1000, 1000)])
        if do_deg:
            @pl.when((s >= 5) & (s < 10))
            def _():
                pltpu.sync_copy(sh.at[pl.ds(DEG0 + (s - 5) * 1000, 1000)],
                                deg_out.at[c].at[pl.ds((s - 5) * 1000, 1000)])

    return k


_sc_layer1 = _sc_layer_kernel(True)
_sc_layer2 = _sc_layer_kernel(False)


def _sc_layer(a0, v_idx3, e_idx3, need_deg, z, ones_h):
    fn = _sc_layer1 if need_deg else _sc_layer2
    x1, msgh, degf = fn(a0, v_idx3, e_idx3, z, ones_h)
    msg = msgh.reshape(N0, H)
    degp = degf.reshape(N0, H)
    return x1, msg, degp


def kernel(x_0, x_1, vertex_idx, hyperedge_idx,
           W0_in, b0_in, W1_in, b1_in, Wl0, bl0, Wl1, bl1,
           Wo0, bo0, Wo1, bo1):
    v_idx3 = vertex_idx.astype(jnp.int32).reshape(NCHUNK, 1, CW)
    e_idx3 = hyperedge_idx.astype(jnp.int32).reshape(NCHUNK, 1, CW)
    z = jnp.zeros((1000, H), jnp.float32)
    ones_h = jnp.ones((CW, H), jnp.float32)

    h0 = _matmul(x_0, W0_in, b0_in)

    a1 = _matmul(h0, Wl0, bl0)
    _, msg1, degp = _sc_layer(a1, v_idx3, e_idx3, True, z, ones_h)
    x0_1, _ = _combine(a1, msg1, degp, relu=True)

    a2 = _matmul(x0_1, Wl1, bl1)
    x1_2, msg2, _ = _sc_layer(a2, v_idx3, e_idx3, False, z, ones_h)
    x0_2, s0 = _combine(a2, msg2, degp, relu=False)

    s1 = _colsum(x1_2, RB1)
    return _head(s0, s1, Wo0, Wo1, bo0, bo1)


# speedup vs baseline: 9.0528x; 2.3403x over previous
import functools

import jax
import jax.numpy as jnp
from jax import lax
from jax.experimental import pallas as pl
from jax.experimental.pallas import tpu as pltpu
from jax.experimental.pallas import tpu_sc as plsc

N0 = 10000
N1 = 20000
E = 640000
H = 128

RB0 = 2000
RB1 = 2000


def _mm_body(x_ref, w_ref, b_ref, o_ref):
    x = x_ref[...].astype(jnp.bfloat16)
    w = w_ref[...].astype(jnp.bfloat16)
    o_ref[...] = (jnp.dot(x, w, preferred_element_type=jnp.float32)
                  + b_ref[...])


def _matmul(x, W, b):
    n, f = x.shape
    return pl.pallas_call(
        _mm_body,
        grid=(n // RB0,),
        in_specs=[
            pl.BlockSpec((RB0, f), lambda i: (i, 0)),
            pl.BlockSpec((f, H), lambda i: (0, 0)),
            pl.BlockSpec((1, H), lambda i: (0, 0)),
        ],
        out_specs=pl.BlockSpec((RB0, H), lambda i: (i, 0)),
        out_shape=jax.ShapeDtypeStruct((n, H), jnp.float32),
    )(x, W, b.reshape(1, H))


def _combine_body(a_ref, m_ref, d_ref, o_ref, s_ref, *, relu):
    i = pl.program_id(0)
    deg = d_ref[0, :, 0:1] + d_ref[1, :, 0:1]
    r = 1.0 / jnp.maximum(deg, 1.0)
    x = a_ref[...] + m_ref[...] * r
    if relu:
        x = jnp.maximum(x, 0.0)
    o_ref[...] = x

    @pl.when(i == 0)
    def _():
        s_ref[...] = jnp.zeros_like(s_ref)
    s_ref[...] += jnp.sum(x, axis=0, keepdims=True)


def _combine(a, msg, degp, relu):
    return pl.pallas_call(
        functools.partial(_combine_body, relu=relu),
        grid=(N0 // RB0,),
        in_specs=[
            pl.BlockSpec((RB0, H), lambda i: (i, 0)),
            pl.BlockSpec((RB0, H), lambda i: (i, 0)),
            pl.BlockSpec((2, RB0, H), lambda i: (0, i, 0)),
        ],
        out_specs=[
            pl.BlockSpec((RB0, H), lambda i: (i, 0)),
            pl.BlockSpec((1, H), lambda i: (0, 0)),
        ],
        out_shape=[
            jax.ShapeDtypeStruct((N0, H), jnp.float32),
            jax.ShapeDtypeStruct((1, H), jnp.float32),
        ],
    )(a, msg, degp)


def _colsum_body(x_ref, s_ref):
    i = pl.program_id(0)

    @pl.when(i == 0)
    def _():
        s_ref[...] = jnp.zeros_like(s_ref)
    s_ref[...] += jnp.sum(x_ref[...], axis=0, keepdims=True)


def _colsum(x, rb):
    n = x.shape[0]
    return pl.pallas_call(
        _colsum_body,
        grid=(n // rb,),
        in_specs=[pl.BlockSpec((rb, H), lambda i: (i, 0))],
        out_specs=pl.BlockSpec((1, H), lambda i: (0, 0)),
        out_shape=jax.ShapeDtypeStruct((1, H), jnp.float32),
    )(x)


def _head_body(s0_ref, s1_ref, w0_ref, w1_ref, b_ref, o_ref):
    w0 = w0_ref[...].astype(jnp.bfloat16).astype(jnp.float32)
    w1 = w1_ref[...].astype(jnp.bfloat16).astype(jnp.float32)
    m0 = jnp.sum(s0_ref[...] * w0) / N0
    m1 = jnp.sum(s1_ref[...] * w1) / N1
    o_ref[...] = (m0 + m1).reshape(1, 1) + b_ref[...]


def _head(s0, s1, Wo0, Wo1, bo0, bo1):
    out = pl.pallas_call(
        _head_body,
        in_specs=[pl.BlockSpec((1, H), lambda: (0, 0))] * 4 +
                 [pl.BlockSpec((1, 1), lambda: (0, 0))],
        out_specs=pl.BlockSpec((1, 1), lambda: (0, 0)),
        out_shape=jax.ShapeDtypeStruct((1, 1), jnp.float32),
    )(s0, s1, Wo0.reshape(1, H), Wo1.reshape(1, H),
      (bo0 + bo1).reshape(1, 1))
    return out.reshape(1)


CW = 128
NCHUNK = E // CW
NTILES = 16
CPT = (NCHUNK + NTILES - 1) // NTILES
X1H = N1 // 2
MSGH = N0 // 2


DCH = NCHUNK // 2


def _deg_kernel():
    mesh = plsc.VectorSubcoreMesh(core_axis_name="c", subcore_axis_name="s")
    out_type = [jax.ShapeDtypeStruct((2, N0, H), jnp.float32)]
    scratch = [
        pltpu.VMEM_SHARED((N0 + 8, H), jnp.float32),
        pltpu.VMEM((1, CW), jnp.int32),
        pltpu.VMEM((1, CW), jnp.int32),
        pltpu.VMEM((CW, H), jnp.float32),
        pltpu.SemaphoreType.DMA,
        pltpu.SemaphoreType.DMA,
        pltpu.SemaphoreType.DMA,
        pltpu.SemaphoreType.DMA,
    ]

    @functools.partial(pl.kernel, out_type=out_type, mesh=mesh,
                       scratch_types=scratch)
    def k(vidx, z, ones_h, deg_out, sh, vb0, vb1, onesbuf, si0, si1,
          ss0, ss1):
        c = lax.axis_index("c")
        s = lax.axis_index("s")
        vb = (vb0, vb1)
        si = (si0, si1)
        ss = (ss0, ss1)
        NIT = DCH // NTILES + 2

        @pl.when(s < 10)
        def _():
            pltpu.sync_copy(z, sh.at[pl.ds(s * 1000, 1000)])
        pltpu.sync_copy(ones_h, onesbuf)
        plsc.subcore_barrier()

        base = c * DCH

        def chunk(i):
            return base + s + i * NTILES

        def vld(i):
            return (s + i * NTILES) < DCH

        pltpu.async_copy(vidx.at[chunk(0)], vb[0], si[0])

        def body(i2, carry):
            for u in range(2):
                i = 2 * i2 + u
                b = u
                o = 1 - u

                @pl.when(vld(i))
                def _():
                    pltpu.make_async_copy(vidx.at[0], vb[b], si[b]).wait()
                    pltpu.async_copy(onesbuf, sh.at[vb[b].at[0]], ss[b],
                                     add=True)

                @pl.when((i >= 1) & vld(i - 1))
                def _():
                    pltpu.make_async_copy(
                        onesbuf, sh.at[pl.ds(0, CW)], ss[o]).wait()

                @pl.when(vld(i + 1))
                def _():
                    pltpu.async_copy(vidx.at[chunk(i + 1)], vb[o], si[o])
            return carry

        lax.fori_loop(0, NIT, body, 0)
        plsc.subcore_barrier()

        @pl.when(s < 10)
        def _():
            pltpu.sync_copy(sh.at[pl.ds(s * 1000, 1000)],
                            deg_out.at[c].at[pl.ds(s * 1000, 1000)])

    return k


def _sc_layer_kernel():
    mesh = plsc.VectorSubcoreMesh(core_axis_name="c", subcore_axis_name="s")
    out_type = [
        jax.ShapeDtypeStruct((N1, H), jnp.float32),
        jax.ShapeDtypeStruct((2, MSGH, H), jnp.float32),
    ]
    scratch = [
        pltpu.VMEM_SHARED((X1H + 8, H), jnp.float32),
        pltpu.VMEM((1, CW), jnp.int32),
        pltpu.VMEM((1, CW), jnp.int32),
        pltpu.VMEM((1, CW), jnp.int32),
        pltpu.VMEM((1, CW), jnp.int32),
        pltpu.VMEM((1, CW), jnp.int32),
        pltpu.VMEM((1, CW), jnp.int32),
        pltpu.VMEM((CW, H), jnp.float32),
        pltpu.VMEM((CW, H), jnp.float32),
    ] + [pltpu.SemaphoreType.DMA] * 8

    @functools.partial(pl.kernel, out_type=out_type, mesh=mesh,
                       scratch_types=scratch)
    def k(a0, vidx, eidx, z, x1_out, msg_out,
          sh, vb0, vb1, eb0, eb1, lb0, lb1, r0, r1,
          sv0, sv1, se0, se1, sg0, sg1, ss0, ss1):
        c = lax.axis_index("c")
        s = lax.axis_index("s")
        vb, eb, lb, rws = (vb0, vb1), (eb0, eb1), (lb0, lb1), (r0, r1)
        sv, se, sg, ss = (sv0, sv1), (se0, se1), (sg0, sg1), (ss0, ss1)
        NIT = CPT // 2 + 1

        def chunk(i):
            return s + i * NTILES

        def vld(i):
            return (s + i * NTILES) < NCHUNK

        def run_pass(table, gsel, lo, size):
            def issue_idx(i, b):
                pltpu.async_copy(vidx.at[chunk(i)], vb[b], sv[b])
                pltpu.async_copy(eidx.at[chunk(i)], eb[b], se[b])

            issue_idx(0, 0)

            def body(i2, carry):
                for u in range(2):
                    i = 2 * i2 + u
                    b = u
                    o = 1 - u
                    gi, sc_i = (vb, eb) if gsel == 0 else (eb, vb)

                    @pl.when((i >= 2) & vld(i - 2))
                    def _():
                        pltpu.make_async_copy(
                            rws[b], sh.at[pl.ds(0, CW)], ss[b]).wait()

                    @pl.when(vld(i))
                    def _():
                        pltpu.make_async_copy(vidx.at[0], vb[b], sv[b]).wait()
                        pltpu.make_async_copy(eidx.at[0], eb[b], se[b]).wait()
                        for t in range(CW // 16):
                            sl = pl.ds(t * 16, 16)
                            lx = sc_i[b][0, sl] - lo
                            ok = (lx >= 0) & (lx < size)
                            lb[b][0, sl] = jnp.where(ok, lx, size + (t % 8))
                        pltpu.async_copy(table.at[gi[b].at[0]], rws[b], sg[b])

                    @pl.when((i >= 1) & vld(i - 1))
                    def _():
                        pltpu.make_async_copy(
                            table.at[pl.ds(0, CW)], rws[o], sg[o]).wait()
                        pltpu.async_copy(rws[o], sh.at[lb[o].at[0]], ss[o],
                                         add=True)

                    @pl.when(vld(i + 1))
                    def _():
                        issue_idx(i + 1, o)
                return carry

            lax.fori_loop(0, NIT, body, 0)
            last = 2 * NIT - 1
            b_last = last % 2

            @pl.when(vld(last))
            def _():
                pltpu.make_async_copy(
                    table.at[pl.ds(0, CW)], rws[b_last], sg[b_last]).wait()
                pltpu.async_copy(rws[b_last], sh.at[lb[b_last].at[0]],
                                 ss[b_last], add=True)
            for b in range(2):
                @pl.when(vld(last - 1 + b))
                def _():
                    pltpu.make_async_copy(
                        rws[b], sh.at[pl.ds(0, CW)], ss[b]).wait()

        @pl.when(s < 10)
        def _():
            pltpu.sync_copy(z, sh.at[pl.ds(s * 1000, 1000)])
        plsc.subcore_barrier()

        run_pass(a0, 0, c * X1H, X1H)
        plsc.subcore_barrier()

        @pl.when(s < 10)
        def _():
            pltpu.sync_copy(sh.at[pl.ds(s * 1000, 1000)],
                            x1_out.at[pl.ds(c * X1H + s * 1000, 1000)])

        @pl.when(s < 5)
        def _():
            pltpu.sync_copy(z, sh.at[pl.ds(s * 1000, 1000)])
        plsc.subcore_barrier()

        run_pass(x1_out, 1, c * MSGH, MSGH)
        plsc.subcore_barrier()

        @pl.when(s < 5)
        def _():
            pltpu.sync_copy(sh.at[pl.ds(s * 1000, 1000)],
                            msg_out.at[c].at[pl.ds(s * 1000, 1000)])

    return k


_sc_deg = _deg_kernel()
_sc_layer_k = _sc_layer_kernel()


def _sc_layer(a0, v_idx3, e_idx3, z):
    x1, msgh = _sc_layer_k(a0, v_idx3, e_idx3, z)
    return x1, msgh.reshape(N0, H)


def kernel(x_0, x_1, vertex_idx, hyperedge_idx,
           W0_in, b0_in, W1_in, b1_in, Wl0, bl0, Wl1, bl1,
           Wo0, bo0, Wo1, bo1):
    v_idx3 = vertex_idx.astype(jnp.int32).reshape(NCHUNK, 1, CW)
    e_idx3 = hyperedge_idx.astype(jnp.int32).reshape(NCHUNK, 1, CW)
    z = jnp.zeros((1000, H), jnp.float32)
    ones_h = jnp.ones((CW, H), jnp.float32)
    degp = _sc_deg(v_idx3, z, ones_h)[0]

    h0 = _matmul(x_0, W0_in, b0_in)

    a1 = _matmul(h0, Wl0, bl0)
    _, msg1 = _sc_layer(a1, v_idx3, e_idx3, z)
    x0_1, _ = _combine(a1, msg1, degp, relu=True)

    a2 = _matmul(x0_1, Wl1, bl1)
    x1_2, msg2 = _sc_layer(a2, v_idx3, e_idx3, z)
    x0_2, s0 = _combine(a2, msg2, degp, relu=False)

    s1 = _colsum(x1_2, RB1)
    return _head(s0, s1, Wo0, Wo1, bo0, bo1)


# speedup vs baseline: 9.1402x; 1.0097x over previous
import functools

import jax
import jax.numpy as jnp
from jax import lax
from jax.experimental import pallas as pl
from jax.experimental.pallas import tpu as pltpu
from jax.experimental.pallas import tpu_sc as plsc

N0 = 10000
N1 = 20000
E = 640000
H = 128

RB0 = 2000
RB1 = 2000


def _mm_body(x_ref, w_ref, b_ref, o_ref):
    x = x_ref[...].astype(jnp.bfloat16)
    w = w_ref[...].astype(jnp.bfloat16)
    o_ref[...] = (jnp.dot(x, w, preferred_element_type=jnp.float32)
                  + b_ref[...])


def _proj_lin_body(x_ref, w0_ref, b0_ref, w1_ref, b1_ref, o_ref):
    x = x_ref[...].astype(jnp.bfloat16)
    w0 = w0_ref[...].astype(jnp.bfloat16)
    h = jnp.dot(x, w0, preferred_element_type=jnp.float32) + b0_ref[...]
    w1 = w1_ref[...].astype(jnp.bfloat16)
    o_ref[...] = (jnp.dot(h.astype(jnp.bfloat16), w1,
                          preferred_element_type=jnp.float32) + b1_ref[...])


def _proj_lin(x, W0, b0, W1, b1):
    n, f = x.shape
    return pl.pallas_call(
        _proj_lin_body,
        grid=(n // RB0,),
        in_specs=[
            pl.BlockSpec((RB0, f), lambda i: (i, 0)),
            pl.BlockSpec((f, H), lambda i: (0, 0)),
            pl.BlockSpec((1, H), lambda i: (0, 0)),
            pl.BlockSpec((H, H), lambda i: (0, 0)),
            pl.BlockSpec((1, H), lambda i: (0, 0)),
        ],
        out_specs=pl.BlockSpec((RB0, H), lambda i: (i, 0)),
        out_shape=jax.ShapeDtypeStruct((n, H), jnp.float32),
    )(x, W0, b0.reshape(1, H), W1, b1.reshape(1, H))


def _combine_mm_body(a_ref, m_ref, d_ref, w_ref, b_ref, o_ref):
    deg = d_ref[0, :, 0:1] + d_ref[1, :, 0:1]
    r = 1.0 / jnp.maximum(deg, 1.0)
    x = jnp.maximum(a_ref[...] + m_ref[...] * r, 0.0)
    w = w_ref[...].astype(jnp.bfloat16)
    o_ref[...] = (jnp.dot(x.astype(jnp.bfloat16), w,
                          preferred_element_type=jnp.float32) + b_ref[...])


def _combine_mm(a, msg, degp, W, b):
    return pl.pallas_call(
        _combine_mm_body,
        grid=(N0 // RB0,),
        in_specs=[
            pl.BlockSpec((RB0, H), lambda i: (i, 0)),
            pl.BlockSpec((RB0, H), lambda i: (i, 0)),
            pl.BlockSpec((2, RB0, H), lambda i: (0, i, 0)),
            pl.BlockSpec((H, H), lambda i: (0, 0)),
            pl.BlockSpec((1, H), lambda i: (0, 0)),
        ],
        out_specs=pl.BlockSpec((RB0, H), lambda i: (i, 0)),
        out_shape=jax.ShapeDtypeStruct((N0, H), jnp.float32),
    )(a, msg, degp, W, b.reshape(1, H))


def _combine_sum_body(a_ref, m_ref, d_ref, s_ref):
    i = pl.program_id(0)
    deg = d_ref[0, :, 0:1] + d_ref[1, :, 0:1]
    r = 1.0 / jnp.maximum(deg, 1.0)
    x = a_ref[...] + m_ref[...] * r

    @pl.when(i == 0)
    def _():
        s_ref[...] = jnp.zeros_like(s_ref)
    s_ref[...] += jnp.sum(x, axis=0, keepdims=True)


def _combine_sum(a, msg, degp):
    return pl.pallas_call(
        _combine_sum_body,
        grid=(N0 // RB0,),
        in_specs=[
            pl.BlockSpec((RB0, H), lambda i: (i, 0)),
            pl.BlockSpec((RB0, H), lambda i: (i, 0)),
            pl.BlockSpec((2, RB0, H), lambda i: (0, i, 0)),
        ],
        out_specs=pl.BlockSpec((1, H), lambda i: (0, 0)),
        out_shape=jax.ShapeDtypeStruct((1, H), jnp.float32),
    )(a, msg, degp)


def _colsum_body(x_ref, s_ref):
    i = pl.program_id(0)

    @pl.when(i == 0)
    def _():
        s_ref[...] = jnp.zeros_like(s_ref)
    s_ref[...] += jnp.sum(x_ref[...], axis=0, keepdims=True)


def _colsum(x, rb):
    n = x.shape[0]
    return pl.pallas_call(
        _colsum_body,
        grid=(n // rb,),
        in_specs=[pl.BlockSpec((rb, H), lambda i: (i, 0))],
        out_specs=pl.BlockSpec((1, H), lambda i: (0, 0)),
        out_shape=jax.ShapeDtypeStruct((1, H), jnp.float32),
    )(x)


def _head_body(s0_ref, s1_ref, w0_ref, w1_ref, b_ref, o_ref):
    w0 = w0_ref[...].astype(jnp.bfloat16).astype(jnp.float32)
    w1 = w1_ref[...].astype(jnp.bfloat16).astype(jnp.float32)
    m0 = jnp.sum(s0_ref[...] * w0) / N0
    m1 = jnp.sum(s1_ref[...] * w1) / N1
    o_ref[...] = (m0 + m1).reshape(1, 1) + b_ref[...]


def _head(s0, s1, Wo0, Wo1, bo0, bo1):
    out = pl.pallas_call(
        _head_body,
        in_specs=[pl.BlockSpec((1, H), lambda: (0, 0))] * 4 +
                 [pl.BlockSpec((1, 1), lambda: (0, 0))],
        out_specs=pl.BlockSpec((1, 1), lambda: (0, 0)),
        out_shape=jax.ShapeDtypeStruct((1, 1), jnp.float32),
    )(s0, s1, Wo0.reshape(1, H), Wo1.reshape(1, H),
      (bo0 + bo1).reshape(1, 1))
    return out.reshape(1)


CW = 128
NCHUNK = E // CW
NTILES = 16
CPT = (NCHUNK + NTILES - 1) // NTILES
X1H = N1 // 2
MSGH = N0 // 2


DCH = NCHUNK // 2


def _deg_kernel():
    mesh = plsc.VectorSubcoreMesh(core_axis_name="c", subcore_axis_name="s")
    out_type = [jax.ShapeDtypeStruct((2, N0, H), jnp.float32)]
    scratch = [
        pltpu.VMEM_SHARED((N0 + 8, H), jnp.float32),
        pltpu.VMEM((1, CW), jnp.int32),
        pltpu.VMEM((1, CW), jnp.int32),
        pltpu.VMEM((CW, H), jnp.float32),
        pltpu.SemaphoreType.DMA,
        pltpu.SemaphoreType.DMA,
        pltpu.SemaphoreType.DMA,
        pltpu.SemaphoreType.DMA,
    ]

    @functools.partial(pl.kernel, out_type=out_type, mesh=mesh,
                       scratch_types=scratch)
    def k(vidx, z, ones_h, deg_out, sh, vb0, vb1, onesbuf, si0, si1,
          ss0, ss1):
        c = lax.axis_index("c")
        s = lax.axis_index("s")
        vb = (vb0, vb1)
        si = (si0, si1)
        ss = (ss0, ss1)
        NIT = DCH // NTILES + 2

        @pl.when(s < 10)
        def _():
            pltpu.sync_copy(z, sh.at[pl.ds(s * 1000, 1000)])
        pltpu.sync_copy(ones_h, onesbuf)
        plsc.subcore_barrier()

        base = c * DCH

        def chunk(i):
            return base + s + i * NTILES

        def vld(i):
            return (s + i * NTILES) < DCH

        pltpu.async_copy(vidx.at[chunk(0)], vb[0], si[0])

        def body(i2, carry):
            for u in range(2):
                i = 2 * i2 + u
                b = u
                o = 1 - u

                @pl.when(vld(i))
                def _():
                    pltpu.make_async_copy(vidx.at[0], vb[b], si[b]).wait()
                    pltpu.async_copy(onesbuf, sh.at[vb[b].at[0]], ss[b],
                                     add=True)

                @pl.when((i >= 1) & vld(i - 1))
                def _():
                    pltpu.make_async_copy(
                        onesbuf, sh.at[pl.ds(0, CW)], ss[o]).wait()

                @pl.when(vld(i + 1))
                def _():
                    pltpu.async_copy(vidx.at[chunk(i + 1)], vb[o], si[o])
            return carry

        lax.fori_loop(0, NIT, body, 0)
        plsc.subcore_barrier()

        @pl.when(s < 10)
        def _():
            pltpu.sync_copy(sh.at[pl.ds(s * 1000, 1000)],
                            deg_out.at[c].at[pl.ds(s * 1000, 1000)])

    return k


def _sc_layer_kernel():
    mesh = plsc.VectorSubcoreMesh(core_axis_name="c", subcore_axis_name="s")
    out_type = [
        jax.ShapeDtypeStruct((N1, H), jnp.float32),
        jax.ShapeDtypeStruct((2, MSGH, H), jnp.float32),
    ]
    scratch = [
        pltpu.VMEM_SHARED((X1H + 8, H), jnp.float32),
        pltpu.VMEM((2, CW), jnp.int32),
        pltpu.VMEM((2, CW), jnp.int32),
        pltpu.VMEM((1, CW), jnp.int32),
        pltpu.VMEM((1, CW), jnp.int32),
        pltpu.VMEM((CW, H), jnp.float32),
        pltpu.VMEM((CW, H), jnp.float32),
    ] + [pltpu.SemaphoreType.DMA] * 6

    @functools.partial(pl.kernel, out_type=out_type, mesh=mesh,
                       scratch_types=scratch)
    def k(a0, midx, z, x1_out, msg_out,
          sh, ib0, ib1, lb0, lb1, r0, r1,
          si0, si1, sg0, sg1, ss0, ss1):
        c = lax.axis_index("c")
        s = lax.axis_index("s")
        ib, lb, rws = (ib0, ib1), (lb0, lb1), (r0, r1)
        si, sg, ss = (si0, si1), (sg0, sg1), (ss0, ss1)
        NIT = CPT // 2 + 1

        def chunk(i):
            return s + i * NTILES

        def vld(i):
            return (s + i * NTILES) < NCHUNK

        def run_pass(table, gsel, lo, size):
            def issue_idx(i, b):
                pltpu.async_copy(midx.at[chunk(i)], ib[b], si[b])

            issue_idx(0, 0)

            def body(i2, carry):
                for u in range(2):
                    i = 2 * i2 + u
                    b = u
                    o = 1 - u
                    grow, srow = (0, 1) if gsel == 0 else (1, 0)

                    @pl.when((i >= 2) & vld(i - 2))
                    def _():
                        pltpu.make_async_copy(
                            rws[b], sh.at[pl.ds(0, CW)], ss[b]).wait()

                    @pl.when(vld(i))
                    def _():
                        pltpu.make_async_copy(midx.at[0], ib[b], si[b]).wait()
                        for t in range(CW // 16):
                            sl = pl.ds(t * 16, 16)
                            lx = ib[b][srow, sl] - lo
                            ok = (lx >= 0) & (lx < size)
                            lb[b][0, sl] = jnp.where(ok, lx, size + (t % 8))
                        pltpu.async_copy(table.at[ib[b].at[grow]], rws[b],
                                         sg[b])

                    @pl.when((i >= 1) & vld(i - 1))
                    def _():
                        pltpu.make_async_copy(
                            table.at[pl.ds(0, CW)], rws[o], sg[o]).wait()
                        pltpu.async_copy(rws[o], sh.at[lb[o].at[0]], ss[o],
                                         add=True)

                    @pl.when(vld(i + 1))
                    def _():
                        issue_idx(i + 1, o)
                return carry

            lax.fori_loop(0, NIT, body, 0)
            last = 2 * NIT - 1
            b_last = last % 2

            @pl.when(vld(last))
            def _():
                pltpu.make_async_copy(
                    table.at[pl.ds(0, CW)], rws[b_last], sg[b_last]).wait()
                pltpu.async_copy(rws[b_last], sh.at[lb[b_last].at[0]],
                                 ss[b_last], add=True)
            for b in range(2):
                @pl.when(vld(last - 1 + b))
                def _():
                    pltpu.make_async_copy(
                        rws[b], sh.at[pl.ds(0, CW)], ss[b]).wait()

        @pl.when(s < 10)
        def _():
            pltpu.sync_copy(z, sh.at[pl.ds(s * 1000, 1000)])
        plsc.subcore_barrier()

        run_pass(a0, 0, c * X1H, X1H)
        plsc.subcore_barrier()

        @pl.when(s < 10)
        def _():
            pltpu.sync_copy(sh.at[pl.ds(s * 1000, 1000)],
                            x1_out.at[pl.ds(c * X1H + s * 1000, 1000)])

        @pl.when(s < 5)
        def _():
            pltpu.sync_copy(z, sh.at[pl.ds(s * 1000, 1000)])
        plsc.subcore_barrier()

        run_pass(x1_out, 1, c * MSGH, MSGH)
        plsc.subcore_barrier()

        @pl.when(s < 5)
        def _():
            pltpu.sync_copy(sh.at[pl.ds(s * 1000, 1000)],
                            msg_out.at[c].at[pl.ds(s * 1000, 1000)])

    return k


_sc_deg = _deg_kernel()
_sc_layer_k = _sc_layer_kernel()


def _sc_layer(a0, m_idx3, z):
    x1, msgh = _sc_layer_k(a0, m_idx3, z)
    return x1, msgh.reshape(N0, H)


def kernel(x_0, x_1, vertex_idx, hyperedge_idx,
           W0_in, b0_in, W1_in, b1_in, Wl0, bl0, Wl1, bl1,
           Wo0, bo0, Wo1, bo1):
    v_idx3 = vertex_idx.astype(jnp.int32).reshape(NCHUNK, 1, CW)
    e_idx3 = hyperedge_idx.astype(jnp.int32).reshape(NCHUNK, 1, CW)
    m_idx3 = jnp.concatenate([v_idx3, e_idx3], axis=1)
    z = jnp.zeros((1000, H), jnp.float32)
    ones_h = jnp.ones((CW, H), jnp.float32)
    degp = _sc_deg(v_idx3, z, ones_h)[0]

    a1 = _proj_lin(x_0, W0_in, b0_in, Wl0, bl0)
    _, msg1 = _sc_layer(a1, m_idx3, z)
    a2 = _combine_mm(a1, msg1, degp, Wl1, bl1)
    x1_2, msg2 = _sc_layer(a2, m_idx3, z)
    s0 = _combine_sum(a2, msg2, degp)

    s1 = _colsum(x1_2, RB1)
    return _head(s0, s1, Wo0, Wo1, bo0, bo1)


# speedup vs baseline: 11.5671x; 1.2655x over previous
import functools

import jax
import jax.numpy as jnp
from jax import lax
from jax.experimental import pallas as pl
from jax.experimental.pallas import tpu as pltpu
from jax.experimental.pallas import tpu_sc as plsc

N0 = 10000
N1 = 20000
E = 640000
H = 128

RB0 = 2000
RB1 = 2000


def _mm_body(x_ref, w_ref, b_ref, o_ref):
    x = x_ref[...].astype(jnp.bfloat16)
    w = w_ref[...].astype(jnp.bfloat16)
    o_ref[...] = (jnp.dot(x, w, preferred_element_type=jnp.float32)
                  + b_ref[...])


def _proj_lin_body(x_ref, w0_ref, b0_ref, w1_ref, b1_ref, o_ref):
    x = x_ref[...].astype(jnp.bfloat16)
    w0 = w0_ref[...].astype(jnp.bfloat16)
    h = jnp.dot(x, w0, preferred_element_type=jnp.float32) + b0_ref[...]
    w1 = w1_ref[...].astype(jnp.bfloat16)
    o_ref[...] = (jnp.dot(h.astype(jnp.bfloat16), w1,
                          preferred_element_type=jnp.float32) + b1_ref[...])


def _proj_lin(x, W0, b0, W1, b1):
    n, f = x.shape
    return pl.pallas_call(
        _proj_lin_body,
        grid=(n // RB0,),
        in_specs=[
            pl.BlockSpec((RB0, f), lambda i: (i, 0)),
            pl.BlockSpec((f, H), lambda i: (0, 0)),
            pl.BlockSpec((1, H), lambda i: (0, 0)),
            pl.BlockSpec((H, H), lambda i: (0, 0)),
            pl.BlockSpec((1, H), lambda i: (0, 0)),
        ],
        out_specs=pl.BlockSpec((RB0, H), lambda i: (i, 0)),
        out_shape=jax.ShapeDtypeStruct((n, H), jnp.float32),
    )(x, W0, b0.reshape(1, H), W1, b1.reshape(1, H))


def _combine_mm_body(a_ref, m_ref, d_ref, w_ref, b_ref, o_ref):
    deg = d_ref[0, :, 0:1] + d_ref[1, :, 0:1]
    r = 1.0 / jnp.maximum(deg, 1.0)
    m = m_ref[0] + m_ref[1]
    x = jnp.maximum(a_ref[...] + m * r, 0.0)
    w = w_ref[...].astype(jnp.bfloat16)
    o_ref[...] = (jnp.dot(x.astype(jnp.bfloat16), w,
                          preferred_element_type=jnp.float32) + b_ref[...])


def _combine_mm(a, msgp, degp, W, b):
    return pl.pallas_call(
        _combine_mm_body,
        grid=(N0 // RB0,),
        in_specs=[
            pl.BlockSpec((RB0, H), lambda i: (i, 0)),
            pl.BlockSpec((2, RB0, H), lambda i: (0, i, 0)),
            pl.BlockSpec((2, RB0, H), lambda i: (0, i, 0)),
            pl.BlockSpec((H, H), lambda i: (0, 0)),
            pl.BlockSpec((1, H), lambda i: (0, 0)),
        ],
        out_specs=pl.BlockSpec((RB0, H), lambda i: (i, 0)),
        out_shape=jax.ShapeDtypeStruct((N0, H), jnp.float32),
    )(a, msgp, degp, W, b.reshape(1, H))


def _combine_sum_body(a_ref, m_ref, d_ref, s_ref):
    i = pl.program_id(0)
    deg = d_ref[0, :, 0:1] + d_ref[1, :, 0:1]
    r = 1.0 / jnp.maximum(deg, 1.0)
    x = a_ref[...] + (m_ref[0] + m_ref[1]) * r

    @pl.when(i == 0)
    def _():
        s_ref[...] = jnp.zeros_like(s_ref)
    s_ref[...] += jnp.sum(x, axis=0, keepdims=True)


def _combine_sum(a, msgp, degp):
    return pl.pallas_call(
        _combine_sum_body,
        grid=(N0 // RB0,),
        in_specs=[
            pl.BlockSpec((RB0, H), lambda i: (i, 0)),
            pl.BlockSpec((2, RB0, H), lambda i: (0, i, 0)),
            pl.BlockSpec((2, RB0, H), lambda i: (0, i, 0)),
        ],
        out_specs=pl.BlockSpec((1, H), lambda i: (0, 0)),
        out_shape=jax.ShapeDtypeStruct((1, H), jnp.float32),
    )(a, msgp, degp)


def _colsum_body(x_ref, s_ref):
    i = pl.program_id(0)

    @pl.when(i == 0)
    def _():
        s_ref[...] = jnp.zeros_like(s_ref)
    s_ref[...] += jnp.sum(x_ref[...], axis=0, keepdims=True)


def _colsum(x, rb):
    n = x.shape[0]
    return pl.pallas_call(
        _colsum_body,
        grid=(n // rb,),
        in_specs=[pl.BlockSpec((rb, H), lambda i: (i, 0))],
        out_specs=pl.BlockSpec((1, H), lambda i: (0, 0)),
        out_shape=jax.ShapeDtypeStruct((1, H), jnp.float32),
    )(x)


def _head_body(s0_ref, s1_ref, w0_ref, w1_ref, b_ref, o_ref):
    w0 = w0_ref[...].astype(jnp.bfloat16).astype(jnp.float32)
    w1 = w1_ref[...].astype(jnp.bfloat16).astype(jnp.float32)
    m0 = jnp.sum(s0_ref[...] * w0) / N0
    m1 = jnp.sum(s1_ref[...] * w1) / N1
    o_ref[...] = (m0 + m1).reshape(1, 1) + b_ref[...]


def _head(s0, s1, Wo0, Wo1, bo0, bo1):
    out = pl.pallas_call(
        _head_body,
        in_specs=[pl.BlockSpec((1, H), lambda: (0, 0))] * 4 +
                 [pl.BlockSpec((1, 1), lambda: (0, 0))],
        out_specs=pl.BlockSpec((1, 1), lambda: (0, 0)),
        out_shape=jax.ShapeDtypeStruct((1, 1), jnp.float32),
    )(s0, s1, Wo0.reshape(1, H), Wo1.reshape(1, H),
      (bo0 + bo1).reshape(1, 1))
    return out.reshape(1)


CW = 128
NCHUNK = E // CW
NTILES = 16
CPT = (NCHUNK + NTILES - 1) // NTILES
X1H = N1 // 2
MSGH = N0 // 2


DCH = NCHUNK // 2


def _deg_kernel():
    mesh = plsc.VectorSubcoreMesh(core_axis_name="c", subcore_axis_name="s")
    out_type = [jax.ShapeDtypeStruct((2, N0, H), jnp.float32)]
    scratch = [
        pltpu.VMEM_SHARED((N0 + 8, H), jnp.float32),
        pltpu.VMEM((1, CW), jnp.int32),
        pltpu.VMEM((1, CW), jnp.int32),
        pltpu.VMEM((CW, H), jnp.float32),
        pltpu.SemaphoreType.DMA,
        pltpu.SemaphoreType.DMA,
        pltpu.SemaphoreType.DMA,
        pltpu.SemaphoreType.DMA,
    ]

    @functools.partial(pl.kernel, out_type=out_type, mesh=mesh,
                       scratch_types=scratch)
    def k(vidx, z, ones_h, deg_out, sh, vb0, vb1, onesbuf, si0, si1,
          ss0, ss1):
        c = lax.axis_index("c")
        s = lax.axis_index("s")
        vb = (vb0, vb1)
        si = (si0, si1)
        ss = (ss0, ss1)
        NIT = ((DCH + NTILES - 1) // NTILES) // 2 + 1

        @pl.when(s < 10)
        def _():
            pltpu.sync_copy(z, sh.at[pl.ds(s * 1000, 1000)])
        pltpu.sync_copy(ones_h, onesbuf)
        plsc.subcore_barrier()

        base = c * DCH

        def chunk(i):
            return base + s + i * NTILES

        def vld(i):
            return (s + i * NTILES) < DCH

        pltpu.async_copy(vidx.at[chunk(0)], vb[0], si[0])

        def body(i2, carry):
            for u in range(2):
                i = 2 * i2 + u
                b = u
                o = 1 - u

                @pl.when(vld(i))
                def _():
                    pltpu.make_async_copy(vidx.at[0], vb[b], si[b]).wait()
                    pltpu.async_copy(onesbuf, sh.at[vb[b].at[0]], ss[b],
                                     add=True)

                @pl.when((i >= 1) & vld(i - 1))
                def _():
                    pltpu.make_async_copy(
                        onesbuf, sh.at[pl.ds(0, CW)], ss[o]).wait()

                @pl.when(vld(i + 1))
                def _():
                    pltpu.async_copy(vidx.at[chunk(i + 1)], vb[o], si[o])
            return carry

        lax.fori_loop(0, NIT, body, 0)
        plsc.subcore_barrier()

        @pl.when(s < 10)
        def _():
            pltpu.sync_copy(sh.at[pl.ds(s * 1000, 1000)],
                            deg_out.at[c].at[pl.ds(s * 1000, 1000)])

    return k


def _sc_layer_kernel():
    mesh = plsc.VectorSubcoreMesh(core_axis_name="c", subcore_axis_name="s")
    out_type = [
        jax.ShapeDtypeStruct((N1, H), jnp.float32),
        jax.ShapeDtypeStruct((2, N0, H), jnp.float32),
    ]
    scratch = [
        pltpu.VMEM_SHARED((X1H + 8, H), jnp.float32),
        pltpu.VMEM((2, CW), jnp.int32),
        pltpu.VMEM((2, CW), jnp.int32),
        pltpu.VMEM((1, CW), jnp.int32),
        pltpu.VMEM((1, CW), jnp.int32),
        pltpu.VMEM((CW, H), jnp.float32),
        pltpu.VMEM((CW, H), jnp.float32),
    ] + [pltpu.SemaphoreType.DMA] * 6

    @functools.partial(pl.kernel, out_type=out_type, mesh=mesh,
                       scratch_types=scratch)
    def k(a0, midx, z, x1_out, msg_out,
          sh, ib0, ib1, lb0, lb1, r0, r1,
          si0, si1, sg0, sg1, ss0, ss1):
        c = lax.axis_index("c")
        s = lax.axis_index("s")
        ib, lb, rws = (ib0, ib1), (lb0, lb1), (r0, r1)
        si, sg, ss = (si0, si1), (sg0, sg1), (ss0, ss1)

        def run_pass(table, gsel, lo, size, base, count):
            nit = ((count + NTILES - 1) // NTILES) // 2 + 1

            def chunk(i):
                return base + s + i * NTILES

            def vld(i):
                return (s + i * NTILES) < count

            def issue_idx(i, b):
                pltpu.async_copy(midx.at[chunk(i)], ib[b], si[b])

            issue_idx(0, 0)

            def body(i2, carry):
                for u in range(2):
                    i = 2 * i2 + u
                    b = u
                    o = 1 - u
                    grow, srow = (0, 1) if gsel == 0 else (1, 0)

                    @pl.when((i >= 2) & vld(i - 2))
                    def _():
                        pltpu.make_async_copy(
                            rws[b], sh.at[pl.ds(0, CW)], ss[b]).wait()

                    @pl.when(vld(i))
                    def _():
                        pltpu.make_async_copy(midx.at[0], ib[b], si[b]).wait()
                        for t in range(CW // 16):
                            sl = pl.ds(t * 16, 16)
                            lx = ib[b][srow, sl] - lo
                            ok = (lx >= 0) & (lx < size)
                            lb[b][0, sl] = jnp.where(ok, lx, size + (t % 8))
                        pltpu.async_copy(table.at[ib[b].at[grow]], rws[b],
                                         sg[b])

                    @pl.when((i >= 1) & vld(i - 1))
                    def _():
                        pltpu.make_async_copy(
                            table.at[pl.ds(0, CW)], rws[o], sg[o]).wait()
                        pltpu.async_copy(rws[o], sh.at[lb[o].at[0]], ss[o],
                                         add=True)

                    @pl.when(vld(i + 1))
                    def _():
                        issue_idx(i + 1, o)
                return carry

            lax.fori_loop(0, nit, body, 0)
            last = 2 * nit - 1
            b_last = last % 2

            @pl.when(vld(last))
            def _():
                pltpu.make_async_copy(
                    table.at[pl.ds(0, CW)], rws[b_last], sg[b_last]).wait()
                pltpu.async_copy(rws[b_last], sh.at[lb[b_last].at[0]],
                                 ss[b_last], add=True)
            for b in range(2):
                @pl.when(vld(last - 1 + b))
                def _():
                    pltpu.make_async_copy(
                        rws[b], sh.at[pl.ds(0, CW)], ss[b]).wait()

        @pl.when(s < 10)
        def _():
            pltpu.sync_copy(z, sh.at[pl.ds(s * 1000, 1000)])
        plsc.subcore_barrier()

        run_pass(a0, 0, c * X1H, X1H, 0, NCHUNK)
        plsc.subcore_barrier()

        @pl.when(s < 10)
        def _():
            pltpu.sync_copy(sh.at[pl.ds(s * 1000, 1000)],
                            x1_out.at[pl.ds(c * X1H + s * 1000, 1000)])

        @pl.when(s < 10)
        def _():
            pltpu.sync_copy(z, sh.at[pl.ds(s * 1000, 1000)])
        plsc.subcore_barrier()

        run_pass(x1_out, 1, 0, N0, c * DCH, DCH)
        plsc.subcore_barrier()

        @pl.when(s < 10)
        def _():
            pltpu.sync_copy(sh.at[pl.ds(s * 1000, 1000)],
                            msg_out.at[c].at[pl.ds(s * 1000, 1000)])

    return k


_sc_deg = _deg_kernel()
_sc_layer_k = _sc_layer_kernel()


def _sc_layer(a0, m_idx3, z):
    return _sc_layer_k(a0, m_idx3, z)


def kernel(x_0, x_1, vertex_idx, hyperedge_idx,
           W0_in, b0_in, W1_in, b1_in, Wl0, bl0, Wl1, bl1,
           Wo0, bo0, Wo1, bo1):
    v_idx3 = vertex_idx.astype(jnp.int32).reshape(NCHUNK, 1, CW)
    e_idx3 = hyperedge_idx.astype(jnp.int32).reshape(NCHUNK, 1, CW)
    m_idx3 = jnp.concatenate([v_idx3, e_idx3], axis=1)
    z = jnp.zeros((1000, H), jnp.float32)
    ones_h = jnp.ones((CW, H), jnp.float32)
    degp = _sc_deg(v_idx3, z, ones_h)[0]

    a1 = _proj_lin(x_0, W0_in, b0_in, Wl0, bl0)
    _, msg1 = _sc_layer(a1, m_idx3, z)
    a2 = _combine_mm(a1, msg1, degp, Wl1, bl1)
    x1_2, msg2 = _sc_layer(a2, m_idx3, z)
    s0 = _combine_sum(a2, msg2, degp)

    s1 = _colsum(x1_2, RB1)
    return _head(s0, s1, Wo0, Wo1, bo0, bo1)


# speedup vs baseline: 11.6968x; 1.0112x over previous
import functools

import jax
import jax.numpy as jnp
from jax import lax
from jax.experimental import pallas as pl
from jax.experimental.pallas import tpu as pltpu
from jax.experimental.pallas import tpu_sc as plsc

N0 = 10000
N1 = 20000
E = 640000
H = 128

RB0 = 2000
RB1 = 2000


def _mm_body(x_ref, w_ref, b_ref, o_ref):
    x = x_ref[...].astype(jnp.bfloat16)
    w = w_ref[...].astype(jnp.bfloat16)
    o_ref[...] = (jnp.dot(x, w, preferred_element_type=jnp.float32)
                  + b_ref[...])


def _proj_lin_body(x_ref, w0_ref, b0_ref, w1_ref, b1_ref, o_ref):
    x = x_ref[...].astype(jnp.bfloat16)
    w0 = w0_ref[...].astype(jnp.bfloat16)
    h = jnp.dot(x, w0, preferred_element_type=jnp.float32) + b0_ref[...]
    w1 = w1_ref[...].astype(jnp.bfloat16)
    o_ref[...] = (jnp.dot(h.astype(jnp.bfloat16), w1,
                          preferred_element_type=jnp.float32) + b1_ref[...])


def _proj_lin(x, W0, b0, W1, b1):
    n, f = x.shape
    return pl.pallas_call(
        _proj_lin_body,
        grid=(n // RB0,),
        in_specs=[
            pl.BlockSpec((RB0, f), lambda i: (i, 0)),
            pl.BlockSpec((f, H), lambda i: (0, 0)),
            pl.BlockSpec((1, H), lambda i: (0, 0)),
            pl.BlockSpec((H, H), lambda i: (0, 0)),
            pl.BlockSpec((1, H), lambda i: (0, 0)),
        ],
        out_specs=pl.BlockSpec((RB0, H), lambda i: (i, 0)),
        out_shape=jax.ShapeDtypeStruct((n, H), jnp.float32),
    )(x, W0, b0.reshape(1, H), W1, b1.reshape(1, H))


def _combine_mm_body(a_ref, m_ref, d_ref, w_ref, b_ref, o_ref):
    deg = d_ref[0, :, 0:1] + d_ref[1, :, 0:1]
    r = 1.0 / jnp.maximum(deg, 1.0)
    m = m_ref[0] + m_ref[1]
    x = jnp.maximum(a_ref[...] + m * r, 0.0)
    w = w_ref[...].astype(jnp.bfloat16)
    o_ref[...] = (jnp.dot(x.astype(jnp.bfloat16), w,
                          preferred_element_type=jnp.float32) + b_ref[...])


def _combine_mm(a, msgp, degp, W, b):
    return pl.pallas_call(
        _combine_mm_body,
        grid=(N0 // RB0,),
        in_specs=[
            pl.BlockSpec((RB0, H), lambda i: (i, 0)),
            pl.BlockSpec((2, RB0, H), lambda i: (0, i, 0)),
            pl.BlockSpec((2, RB0, H), lambda i: (0, i, 0)),
            pl.BlockSpec((H, H), lambda i: (0, 0)),
            pl.BlockSpec((1, H), lambda i: (0, 0)),
        ],
        out_specs=pl.BlockSpec((RB0, H), lambda i: (i, 0)),
        out_shape=jax.ShapeDtypeStruct((N0, H), jnp.float32),
    )(a, msgp, degp, W, b.reshape(1, H))


def _combine_sum_body(a_ref, m_ref, d_ref, s_ref):
    i = pl.program_id(0)
    deg = d_ref[0, :, 0:1] + d_ref[1, :, 0:1]
    r = 1.0 / jnp.maximum(deg, 1.0)
    x = a_ref[...] + (m_ref[0] + m_ref[1]) * r

    @pl.when(i == 0)
    def _():
        s_ref[...] = jnp.zeros_like(s_ref)
    s_ref[...] += jnp.sum(x, axis=0, keepdims=True)


def _combine_sum(a, msgp, degp):
    return pl.pallas_call(
        _combine_sum_body,
        grid=(N0 // RB0,),
        in_specs=[
            pl.BlockSpec((RB0, H), lambda i: (i, 0)),
            pl.BlockSpec((2, RB0, H), lambda i: (0, i, 0)),
            pl.BlockSpec((2, RB0, H), lambda i: (0, i, 0)),
        ],
        out_specs=pl.BlockSpec((1, H), lambda i: (0, 0)),
        out_shape=jax.ShapeDtypeStruct((1, H), jnp.float32),
    )(a, msgp, degp)


def _colsum_body(x_ref, s_ref):
    i = pl.program_id(0)

    @pl.when(i == 0)
    def _():
        s_ref[...] = jnp.zeros_like(s_ref)
    s_ref[...] += jnp.sum(x_ref[...], axis=0, keepdims=True)


def _colsum(x, rb):
    n = x.shape[0]
    return pl.pallas_call(
        _colsum_body,
        grid=(n // rb,),
        in_specs=[pl.BlockSpec((rb, H), lambda i: (i, 0))],
        out_specs=pl.BlockSpec((1, H), lambda i: (0, 0)),
        out_shape=jax.ShapeDtypeStruct((1, H), jnp.float32),
    )(x)


def _head_body(s0_ref, s1_ref, w0_ref, w1_ref, b_ref, o_ref):
    w0 = w0_ref[...].astype(jnp.bfloat16).astype(jnp.float32)
    w1 = w1_ref[...].astype(jnp.bfloat16).astype(jnp.float32)
    m0 = jnp.sum(s0_ref[...] * w0) / N0
    m1 = jnp.sum(s1_ref[...] * w1) / N1
    o_ref[...] = (m0 + m1).reshape(1, 1) + b_ref[...]


def _head(s0, s1, Wo0, Wo1, bo0, bo1):
    out = pl.pallas_call(
        _head_body,
        in_specs=[pl.BlockSpec((1, H), lambda: (0, 0))] * 4 +
                 [pl.BlockSpec((1, 1), lambda: (0, 0))],
        out_specs=pl.BlockSpec((1, 1), lambda: (0, 0)),
        out_shape=jax.ShapeDtypeStruct((1, 1), jnp.float32),
    )(s0, s1, Wo0.reshape(1, H), Wo1.reshape(1, H),
      (bo0 + bo1).reshape(1, 1))
    return out.reshape(1)


CW = 128
NCHUNK = E // CW
NTILES = 16
CPT = (NCHUNK + NTILES - 1) // NTILES
X1H = N1 // 2
MSGH = N0 // 2


DCH = NCHUNK // 2


def _deg_kernel():
    mesh = plsc.VectorSubcoreMesh(core_axis_name="c", subcore_axis_name="s")
    out_type = [jax.ShapeDtypeStruct((2, N0, H), jnp.float32)]
    scratch = [
        pltpu.VMEM_SHARED((N0 + 8, H), jnp.float32),
        pltpu.VMEM((1, CW), jnp.int32),
        pltpu.VMEM((1, CW), jnp.int32),
        pltpu.VMEM((CW, H), jnp.float32),
        pltpu.SemaphoreType.DMA,
        pltpu.SemaphoreType.DMA,
        pltpu.SemaphoreType.DMA,
        pltpu.SemaphoreType.DMA,
    ]

    @functools.partial(pl.kernel, out_type=out_type, mesh=mesh,
                       scratch_types=scratch)
    def k(vidx, z, ones_h, deg_out, sh, vb0, vb1, onesbuf, si0, si1,
          ss0, ss1):
        c = lax.axis_index("c")
        s = lax.axis_index("s")
        vb = (vb0, vb1)
        si = (si0, si1)
        ss = (ss0, ss1)
        NIT = ((DCH + NTILES - 1) // NTILES) // 2 + 1

        @pl.when(s < 10)
        def _():
            pltpu.sync_copy(z, sh.at[pl.ds(s * 1000, 1000)])
        pltpu.sync_copy(ones_h, onesbuf)
        plsc.subcore_barrier()

        base = c * DCH

        def chunk(i):
            return base + s + i * NTILES

        def vld(i):
            return (s + i * NTILES) < DCH

        pltpu.async_copy(vidx.at[chunk(0)], vb[0], si[0])

        def body(i2, carry):
            for u in range(2):
                i = 2 * i2 + u
                b = u
                o = 1 - u

                @pl.when(vld(i))
                def _():
                    pltpu.make_async_copy(vidx.at[0], vb[b], si[b]).wait()
                    pltpu.async_copy(onesbuf, sh.at[vb[b].at[0]], ss[b],
                                     add=True)

                @pl.when((i >= 1) & vld(i - 1))
                def _():
                    pltpu.make_async_copy(
                        onesbuf, sh.at[pl.ds(0, CW)], ss[o]).wait()

                @pl.when(vld(i + 1))
                def _():
                    pltpu.async_copy(vidx.at[chunk(i + 1)], vb[o], si[o])
            return carry

        lax.fori_loop(0, NIT, body, 0)
        plsc.subcore_barrier()

        @pl.when(s < 10)
        def _():
            pltpu.sync_copy(sh.at[pl.ds(s * 1000, 1000)],
                            deg_out.at[c].at[pl.ds(s * 1000, 1000)])

    return k


def _sc_layer_kernel():
    mesh = plsc.VectorSubcoreMesh(core_axis_name="c", subcore_axis_name="s")
    out_type = [
        jax.ShapeDtypeStruct((N1, H), jnp.float32),
        jax.ShapeDtypeStruct((2, N0, H), jnp.float32),
    ]
    scratch = [
        pltpu.VMEM_SHARED((X1H + 8, H), jnp.float32),
    ] + [pltpu.VMEM((2, CW), jnp.int32)] * 4 \
      + [pltpu.VMEM((1, CW), jnp.int32)] * 4 \
      + [pltpu.VMEM((CW, H), jnp.float32)] * 2 \
      + [pltpu.SemaphoreType.DMA] * 8

    @functools.partial(pl.kernel, out_type=out_type, mesh=mesh,
                       scratch_types=scratch)
    def k(a0, midx, z, x1_out, msg_out,
          sh, ib0, ib1, ib2, ib3, lb0, lb1, lb2, lb3, r0, r1,
          si0, si1, si2, si3, sg0, sg1, ss0, ss1):
        c = lax.axis_index("c")
        s = lax.axis_index("s")
        ib, lb = (ib0, ib1, ib2, ib3), (lb0, lb1, lb2, lb3)
        rws = (r0, r1)
        si = (si0, si1, si2, si3)
        sg, ss = (sg0, sg1), (ss0, ss1)

        def run_pass(table, gsel, lo, size, base, count):
            cpt = (count + NTILES - 1) // NTILES
            nit = (cpt + 2 + 3) // 4

            def chunk(i):
                return base + s + i * NTILES

            def vld(i):
                return (s + i * NTILES) < count

            def issue_idx(i, q):
                pltpu.async_copy(midx.at[chunk(i)], ib[q], si[q])

            grow, srow = (0, 1) if gsel == 0 else (1, 0)
            for p in range(3):
                @pl.when(vld(p))
                def _():
                    issue_idx(p, p)

            def body(i4, carry):
                for u in range(4):
                    i = 4 * i4 + u
                    qb = u
                    q3 = (u + 3) % 4
                    rb = u % 2
                    ro = 1 - rb

                    @pl.when((i >= 2) & vld(i - 2))
                    def _():
                        pltpu.make_async_copy(
                            rws[rb], sh.at[pl.ds(0, CW)], ss[rb]).wait()

                    @pl.when(vld(i))
                    def _():
                        pltpu.make_async_copy(midx.at[0], ib[qb],
                                              si[qb]).wait()
                        for t in range(CW // 16):
                            sl = pl.ds(t * 16, 16)
                            lx = ib[qb][srow, sl] - lo
                            ok = (lx >= 0) & (lx < size)
                            lb[qb][0, sl] = jnp.where(ok, lx, size + (t % 8))
                        pltpu.async_copy(table.at[ib[qb].at[grow]], rws[rb],
                                         sg[rb])

                    @pl.when((i >= 1) & vld(i - 1))
                    def _():
                        pltpu.make_async_copy(
                            table.at[pl.ds(0, CW)], rws[ro], sg[ro]).wait()
                        pltpu.async_copy(rws[ro], sh.at[lb[q3].at[0]],
                                         ss[ro], add=True)

                    @pl.when(vld(i + 3))
                    def _():
                        issue_idx(i + 3, q3)
                return carry

            lax.fori_loop(0, nit, body, 0)

        @pl.when(s < 10)
        def _():
            pltpu.sync_copy(z, sh.at[pl.ds(s * 1000, 1000)])
        plsc.subcore_barrier()

        run_pass(a0, 0, c * X1H, X1H, 0, NCHUNK)
        plsc.subcore_barrier()

        @pl.when(s < 10)
        def _():
            pltpu.sync_copy(sh.at[pl.ds(s * 1000, 1000)],
                            x1_out.at[pl.ds(c * X1H + s * 1000, 1000)])

        @pl.when(s < 10)
        def _():
            pltpu.sync_copy(z, sh.at[pl.ds(s * 1000, 1000)])
        plsc.subcore_barrier()

        run_pass(x1_out, 1, 0, N0, c * DCH, DCH)
        plsc.subcore_barrier()

        @pl.when(s < 10)
        def _():
            pltpu.sync_copy(sh.at[pl.ds(s * 1000, 1000)],
                            msg_out.at[c].at[pl.ds(s * 1000, 1000)])

    return k


_sc_deg = _deg_kernel()
_sc_layer_k = _sc_layer_kernel()


def _sc_layer(a0, m_idx3, z):
    return _sc_layer_k(a0, m_idx3, z)


def kernel(x_0, x_1, vertex_idx, hyperedge_idx,
           W0_in, b0_in, W1_in, b1_in, Wl0, bl0, Wl1, bl1,
           Wo0, bo0, Wo1, bo1):
    v_idx3 = vertex_idx.astype(jnp.int32).reshape(NCHUNK, 1, CW)
    e_idx3 = hyperedge_idx.astype(jnp.int32).reshape(NCHUNK, 1, CW)
    m_idx3 = jnp.concatenate([v_idx3, e_idx3], axis=1)
    z = jnp.zeros((1000, H), jnp.float32)
    ones_h = jnp.ones((CW, H), jnp.float32)
    degp = _sc_deg(v_idx3, z, ones_h)[0]

    a1 = _proj_lin(x_0, W0_in, b0_in, Wl0, bl0)
    _, msg1 = _sc_layer(a1, m_idx3, z)
    a2 = _combine_mm(a1, msg1, degp, Wl1, bl1)
    x1_2, msg2 = _sc_layer(a2, m_idx3, z)
    s0 = _combine_sum(a2, msg2, degp)

    s1 = _colsum(x1_2, RB1)
    return _head(s0, s1, Wo0, Wo1, bo0, bo1)


# speedup vs baseline: 11.7068x; 1.0009x over previous
import functools

import jax
import jax.numpy as jnp
from jax import lax
from jax.experimental import pallas as pl
from jax.experimental.pallas import tpu as pltpu
from jax.experimental.pallas import tpu_sc as plsc

N0 = 10000
N1 = 20000
E = 640000
H = 128

RB0 = 2000
RB1 = 2000


def _mm_body(x_ref, w_ref, b_ref, o_ref):
    x = x_ref[...].astype(jnp.bfloat16)
    w = w_ref[...].astype(jnp.bfloat16)
    o_ref[...] = (jnp.dot(x, w, preferred_element_type=jnp.float32)
                  + b_ref[...])


def _proj_lin_body(x_ref, w0_ref, b0_ref, w1_ref, b1_ref, o_ref):
    x = x_ref[...].astype(jnp.bfloat16)
    w0 = w0_ref[...].astype(jnp.bfloat16)
    h = jnp.dot(x, w0, preferred_element_type=jnp.float32) + b0_ref[...]
    w1 = w1_ref[...].astype(jnp.bfloat16)
    o_ref[...] = (jnp.dot(h.astype(jnp.bfloat16), w1,
                          preferred_element_type=jnp.float32) + b1_ref[...])


def _proj_lin(x, W0, b0, W1, b1):
    n, f = x.shape
    return pl.pallas_call(
        _proj_lin_body,
        grid=(n // RB0,),
        in_specs=[
            pl.BlockSpec((RB0, f), lambda i: (i, 0)),
            pl.BlockSpec((f, H), lambda i: (0, 0)),
            pl.BlockSpec((1, H), lambda i: (0, 0)),
            pl.BlockSpec((H, H), lambda i: (0, 0)),
            pl.BlockSpec((1, H), lambda i: (0, 0)),
        ],
        out_specs=pl.BlockSpec((RB0, H), lambda i: (i, 0)),
        out_shape=jax.ShapeDtypeStruct((n, H), jnp.float32),
    )(x, W0, b0.reshape(1, H), W1, b1.reshape(1, H))


def _combine_mm_body(a_ref, m_ref, d_ref, w_ref, b_ref, o_ref):
    deg = d_ref[0, :, 0:1] + d_ref[1, :, 0:1]
    r = 1.0 / jnp.maximum(deg, 1.0)
    m = m_ref[0] + m_ref[1]
    x = jnp.maximum(a_ref[...] + m * r, 0.0)
    w = w_ref[...].astype(jnp.bfloat16)
    o_ref[...] = (jnp.dot(x.astype(jnp.bfloat16), w,
                          preferred_element_type=jnp.float32) + b_ref[...])


def _combine_mm(a, msgp, degp, W, b):
    return pl.pallas_call(
        _combine_mm_body,
        grid=(N0 // RB0,),
        in_specs=[
            pl.BlockSpec((RB0, H), lambda i: (i, 0)),
            pl.BlockSpec((2, RB0, H), lambda i: (0, i, 0)),
            pl.BlockSpec((2, RB0, H), lambda i: (0, i, 0)),
            pl.BlockSpec((H, H), lambda i: (0, 0)),
            pl.BlockSpec((1, H), lambda i: (0, 0)),
        ],
        out_specs=pl.BlockSpec((RB0, H), lambda i: (i, 0)),
        out_shape=jax.ShapeDtypeStruct((N0, H), jnp.float32),
    )(a, msgp, degp, W, b.reshape(1, H))


def _combine_sum_body(a_ref, m_ref, d_ref, s_ref):
    i = pl.program_id(0)
    deg = d_ref[0, :, 0:1] + d_ref[1, :, 0:1]
    r = 1.0 / jnp.maximum(deg, 1.0)
    x = a_ref[...] + (m_ref[0] + m_ref[1]) * r

    @pl.when(i == 0)
    def _():
        s_ref[...] = jnp.zeros_like(s_ref)
    s_ref[...] += jnp.sum(x, axis=0, keepdims=True)


def _combine_sum(a, msgp, degp):
    return pl.pallas_call(
        _combine_sum_body,
        grid=(N0 // RB0,),
        in_specs=[
            pl.BlockSpec((RB0, H), lambda i: (i, 0)),
            pl.BlockSpec((2, RB0, H), lambda i: (0, i, 0)),
            pl.BlockSpec((2, RB0, H), lambda i: (0, i, 0)),
        ],
        out_specs=pl.BlockSpec((1, H), lambda i: (0, 0)),
        out_shape=jax.ShapeDtypeStruct((1, H), jnp.float32),
    )(a, msgp, degp)


def _colsum_body(x_ref, s_ref):
    i = pl.program_id(0)

    @pl.when(i == 0)
    def _():
        s_ref[...] = jnp.zeros_like(s_ref)
    s_ref[...] += jnp.sum(x_ref[...], axis=0, keepdims=True)


def _colsum(x, rb):
    n = x.shape[0]
    return pl.pallas_call(
        _colsum_body,
        grid=(n // rb,),
        in_specs=[pl.BlockSpec((rb, H), lambda i: (i, 0))],
        out_specs=pl.BlockSpec((1, H), lambda i: (0, 0)),
        out_shape=jax.ShapeDtypeStruct((1, H), jnp.float32),
    )(x)


def _head_body(s0_ref, s1_ref, w0_ref, w1_ref, b_ref, o_ref):
    w0 = w0_ref[...].astype(jnp.bfloat16).astype(jnp.float32)
    w1 = w1_ref[...].astype(jnp.bfloat16).astype(jnp.float32)
    m0 = jnp.sum(s0_ref[...] * w0) / N0
    m1 = jnp.sum(s1_ref[...] * w1) / N1
    o_ref[...] = (m0 + m1).reshape(1, 1) + b_ref[...]


def _head(s0, s1, Wo0, Wo1, bo0, bo1):
    out = pl.pallas_call(
        _head_body,
        in_specs=[pl.BlockSpec((1, H), lambda: (0, 0))] * 4 +
                 [pl.BlockSpec((1, 1), lambda: (0, 0))],
        out_specs=pl.BlockSpec((1, 1), lambda: (0, 0)),
        out_shape=jax.ShapeDtypeStruct((1, 1), jnp.float32),
    )(s0, s1, Wo0.reshape(1, H), Wo1.reshape(1, H),
      (bo0 + bo1).reshape(1, 1))
    return out.reshape(1)


CW = 128
NCHUNK = E // CW
NTILES = 16
CPT = (NCHUNK + NTILES - 1) // NTILES
X1H = N1 // 2
MSGH = N0 // 2


DCH = NCHUNK // 2


def _deg_kernel():
    mesh = plsc.VectorSubcoreMesh(core_axis_name="c", subcore_axis_name="s")
    out_type = [jax.ShapeDtypeStruct((2, N0, H), jnp.float32)]
    scratch = [
        pltpu.VMEM_SHARED((N0 + 8, H), jnp.float32),
    ] + [pltpu.VMEM((1, CW), jnp.int32)] * 8 \
      + [pltpu.VMEM((CW, H), jnp.float32)] \
      + [pltpu.SemaphoreType.DMA] * 8

    @functools.partial(pl.kernel, out_type=out_type, mesh=mesh,
                       scratch_types=scratch)
    def k(vidx, z, ones_h, deg_out, sh,
          vb0, vb1, vb2, vb3, db0, db1, db2, db3, onesbuf,
          si0, si1, si2, si3, ss0, ss1, ss2, ss3):
        c = lax.axis_index("c")
        s = lax.axis_index("s")
        vb = (vb0, vb1, vb2, vb3)
        db = (db0, db1, db2, db3)
        si = (si0, si1, si2, si3)
        ss = (ss0, ss1, ss2, ss3)
        cpt = (DCH + NTILES - 1) // NTILES
        NIT = (cpt + 4 + 3) // 4

        @pl.when(s < 10)
        def _():
            pltpu.sync_copy(z, sh.at[pl.ds(s * 1000, 1000)])
        pltpu.sync_copy(ones_h, onesbuf)
        plsc.subcore_barrier()

        base = c * DCH

        def chunk(i):
            return base + s + i * NTILES

        def vld(i):
            return (s + i * NTILES) < DCH

        def issue_idx(i, q):
            pltpu.async_copy(vidx.at[chunk(i)], vb[q], si[q])

        for p in range(3):
            @pl.when(vld(p))
            def _():
                issue_idx(p, p)

        def body(i4, carry):
            for u in range(4):
                i = 4 * i4 + u
                qb = u
                q3 = (u + 3) % 4

                @pl.when((i >= 4) & vld(i - 4))
                def _():
                    pltpu.make_async_copy(
                        onesbuf, sh.at[pl.ds(0, CW)], ss[qb]).wait()

                @pl.when(vld(i))
                def _():
                    pltpu.make_async_copy(vidx.at[0], vb[qb], si[qb]).wait()
                    for t in range(CW // 16):
                        sl = pl.ds(t * 16, 16)
                        db[qb][0, sl] = vb[qb][0, sl]
                    pltpu.async_copy(onesbuf, sh.at[db[qb].at[0]], ss[qb],
                                     add=True)

                @pl.when(vld(i + 3))
                def _():
                    issue_idx(i + 3, q3)
            return carry

        lax.fori_loop(0, NIT, body, 0)
        plsc.subcore_barrier()

        @pl.when(s < 10)
        def _():
            pltpu.sync_copy(sh.at[pl.ds(s * 1000, 1000)],
                            deg_out.at[c].at[pl.ds(s * 1000, 1000)])

    return k


def _sc_layer_kernel():
    mesh = plsc.VectorSubcoreMesh(core_axis_name="c", subcore_axis_name="s")
    out_type = [
        jax.ShapeDtypeStruct((N1, H), jnp.float32),
        jax.ShapeDtypeStruct((2, N0, H), jnp.float32),
    ]
    scratch = [
        pltpu.VMEM_SHARED((X1H + 8, H), jnp.float32),
    ] + [pltpu.VMEM((2, CW), jnp.int32)] * 4 \
      + [pltpu.VMEM((1, CW), jnp.int32)] * 4 \
      + [pltpu.VMEM((CW, H), jnp.float32)] * 2 \
      + [pltpu.SemaphoreType.DMA] * 8

    @functools.partial(pl.kernel, out_type=out_type, mesh=mesh,
                       scratch_types=scratch)
    def k(a0, midx, z, x1_out, msg_out,
          sh, ib0, ib1, ib2, ib3, lb0, lb1, lb2, lb3, r0, r1,
          si0, si1, si2, si3, sg0, sg1, ss0, ss1):
        c = lax.axis_index("c")
        s = lax.axis_index("s")
        ib, lb = (ib0, ib1, ib2, ib3), (lb0, lb1, lb2, lb3)
        rws = (r0, r1)
        si = (si0, si1, si2, si3)
        sg, ss = (sg0, sg1), (ss0, ss1)

        def run_pass(table, gsel, lo, size, base, count):
            cpt = (count + NTILES - 1) // NTILES
            nit = (cpt + 2 + 3) // 4

            def chunk(i):
                return base + s + i * NTILES

            def vld(i):
                return (s + i * NTILES) < count

            def issue_idx(i, q):
                pltpu.async_copy(midx.at[chunk(i)], ib[q], si[q])

            grow, srow = (0, 1) if gsel == 0 else (1, 0)
            for p in range(3):
                @pl.when(vld(p))
                def _():
                    issue_idx(p, p)

            def body(i4, carry):
                for u in range(4):
                    i = 4 * i4 + u
                    qb = u
                    q3 = (u + 3) % 4
                    rb = u % 2
                    ro = 1 - rb

                    @pl.when((i >= 2) & vld(i - 2))
                    def _():
                        pltpu.make_async_copy(
                            rws[rb], sh.at[pl.ds(0, CW)], ss[rb]).wait()

                    @pl.when(vld(i))
                    def _():
                        pltpu.make_async_copy(midx.at[0], ib[qb],
                                              si[qb]).wait()
                        for t in range(CW // 16):
                            sl = pl.ds(t * 16, 16)
                            lx = ib[qb][srow, sl] - lo
                            ok = (lx >= 0) & (lx < size)
                            lb[qb][0, sl] = jnp.where(ok, lx, size + (t % 8))
                        pltpu.async_copy(table.at[ib[qb].at[grow]], rws[rb],
                                         sg[rb])

                    @pl.when((i >= 1) & vld(i - 1))
                    def _():
                        pltpu.make_async_copy(
                            table.at[pl.ds(0, CW)], rws[ro], sg[ro]).wait()
                        pltpu.async_copy(rws[ro], sh.at[lb[q3].at[0]],
                                         ss[ro], add=True)

                    @pl.when(vld(i + 3))
                    def _():
                        issue_idx(i + 3, q3)
                return carry

            lax.fori_loop(0, nit, body, 0)

        @pl.when(s < 10)
        def _():
            pltpu.sync_copy(z, sh.at[pl.ds(s * 1000, 1000)])
        plsc.subcore_barrier()

        run_pass(a0, 0, c * X1H, X1H, 0, NCHUNK)
        plsc.subcore_barrier()

        @pl.when(s < 10)
        def _():
            pltpu.sync_copy(sh.at[pl.ds(s * 1000, 1000)],
                            x1_out.at[pl.ds(c * X1H + s * 1000, 1000)])

        @pl.when(s < 10)
        def _():
            pltpu.sync_copy(z, sh.at[pl.ds(s * 1000, 1000)])
        plsc.subcore_barrier()

        run_pass(x1_out, 1, 0, N0, c * DCH, DCH)
        plsc.subcore_barrier()

        @pl.when(s < 10)
        def _():
            pltpu.sync_copy(sh.at[pl.ds(s * 1000, 1000)],
                            msg_out.at[c].at[pl.ds(s * 1000, 1000)])

    return k


_sc_deg = _deg_kernel()
_sc_layer_k = _sc_layer_kernel()


def _sc_layer(a0, m_idx3, z):
    return _sc_layer_k(a0, m_idx3, z)


def kernel(x_0, x_1, vertex_idx, hyperedge_idx,
           W0_in, b0_in, W1_in, b1_in, Wl0, bl0, Wl1, bl1,
           Wo0, bo0, Wo1, bo1):
    v_idx3 = vertex_idx.astype(jnp.int32).reshape(NCHUNK, 1, CW)
    e_idx3 = hyperedge_idx.astype(jnp.int32).reshape(NCHUNK, 1, CW)
    m_idx3 = jnp.concatenate([v_idx3, e_idx3], axis=1)
    z = jnp.zeros((1000, H), jnp.float32)
    ones_h = jnp.ones((CW, H), jnp.float32)
    degp = _sc_deg(v_idx3, z, ones_h)[0]

    a1 = _proj_lin(x_0, W0_in, b0_in, Wl0, bl0)
    _, msg1 = _sc_layer(a1, m_idx3, z)
    a2 = _combine_mm(a1, msg1, degp, Wl1, bl1)
    x1_2, msg2 = _sc_layer(a2, m_idx3, z)
    s0 = _combine_sum(a2, msg2, degp)

    s1 = _colsum(x1_2, RB1)
    return _head(s0, s1, Wo0, Wo1, bo0, bo1)
